# baseline (speedup 1.0000x reference)
import functools
import jax
import jax.numpy as jnp
from jax import lax
from jax.experimental import pallas as pl
from jax.experimental.pallas import tpu as pltpu, tpu_sc as plsc

VOCAB = 32768
ROWS = 2048
NC = 2
NS = 16
NW = NC * NS
RPW = ROWS // NW
CHUNK = 4096
NCH = RPW * (VOCAB // CHUNK) // 8

_mesh = plsc.VectorSubcoreMesh(core_axis_name="c", subcore_axis_name="s")


@functools.partial(
    pl.kernel,
    out_type=jax.ShapeDtypeStruct((ROWS, VOCAB), jnp.float32),
    mesh=_mesh,
    scratch_types=[
        pltpu.VMEM((8, CHUNK), jnp.float32),
        pltpu.VMEM((8, CHUNK), jnp.float32),
        pltpu.VMEM((RPW,), jnp.int32),
        pltpu.SemaphoreType.DMA,
        pltpu.SemaphoreType.DMA,
    ],
    compiler_params=pltpu.CompilerParams(use_tc_tiling_on_sc=True, needs_layout_passes=False),
)
def _sc_onehot(ids_hbm, out_hbm, buf0, buf1, ids_v, sem0, sem1):
    wid = lax.axis_index("s") * NC + lax.axis_index("c")
    base_row = wid * RPW
    bufs = (buf0, buf1)
    sems = (sem0, sem1)

    neg16 = jnp.full((16,), -1000.0, dtype=jnp.float32)
    zero16 = jnp.zeros((16,), dtype=jnp.float32)
    lane = lax.iota(jnp.int32, 16)
    sl16 = lane & 7

    def fill_body(j, _):
        for r in range(8):
            for b in range(2):
                bufs[b][r, pl.ds(j * 16, 16)] = neg16
        return 0

    lax.fori_loop(0, CHUNK // 16, fill_body, 0)

    pltpu.sync_copy(ids_hbm.at[pl.ds(base_row, RPW)], ids_v)

    def chunk_info(t):
        rb = t >> 3
        cc = t & 7
        ids8 = plsc.load_gather(ids_v, [rb * 8 + sl16])
        col = (ids8 + 1) & (VOCAB - 1)
        mask = (lane < 8) & ((col >> 12) == cc)
        return rb, cc, mask, col & (CHUNK - 1)

    def fire(t, b):
        rb, cc, mask, cl = chunk_info(t)
        plsc.store_scatter(bufs[b], [sl16, cl], zero16, mask=mask)
        pltpu.async_copy(
            bufs[b],
            out_hbm.at[
                pl.ds(base_row + rb * 8, 8), pl.ds(cc * CHUNK, CHUNK)
            ],
            sems[b],
        )

    def drain(t, b):
        rb, cc, _, _ = chunk_info(t)
        pltpu.make_async_copy(
            bufs[b],
            out_hbm.at[
                pl.ds(base_row + rb * 8, 8), pl.ds(cc * CHUNK, CHUNK)
            ],
            sems[b],
        ).wait()

    def restore(t, b):
        _, _, mask, cl = chunk_info(t)
        plsc.store_scatter(bufs[b], [sl16, cl], neg16, mask=mask)

    fire(0, 0)
    fire(1, 1)

    def step(o, _):
        for b in range(2):
            t = 2 * o + b
            drain(t - 2, b)
            restore(t - 2, b)
            fire(t, b)
        return 0

    lax.fori_loop(1, NCH // 2, step, 0)

    drain(NCH - 2, 0)
    drain(NCH - 1, 1)


def kernel(input_ids, anchor):
    batch, seq_len = input_ids.shape
    ids_flat = input_ids.reshape(batch * seq_len).astype(jnp.int32)
    out = _sc_onehot(ids_flat)
    return out.reshape(batch, seq_len, VOCAB).astype(anchor.dtype)
